# Initial kernel scaffold; baseline (speedup 1.0000x reference)
import functools

import jax
import jax.numpy as jnp
from jax import lax
from jax.experimental import pallas as pl
from jax.experimental.pallas import tpu as pltpu
from jax.experimental.pallas import tpu_sc as plsc

F = 128
B = 2
L = 8
N = 10000
E = 160000
NA = 2
NC = 2
NS = 16
CHUNK = 128
C_CHUNKS = -(-E // (NS * CHUNK))
E_PAD = NS * CHUNK * C_CHUNKS
N_PAD = 10016
FR = N_PAD // NS
ZR = FR // 2


def _fold_weights(W):
    O = W.shape[1]
    Wm = W.reshape(5, 2 * F, O)
    Wh, Wx = Wm[:, :F, :], Wm[:, F:, :]

    def fold(Wp):
        W0, W1, W2, W3, W4 = (Wp[i] for i in range(5))
        return jnp.stack([W0 - W2 - W4, W1, 2.0 * W2, W3, 2.0 * W4])

    return fold(Wh), fold(Wx)


def _prep_edges(A):
    src = A[0].astype(jnp.int32)
    dst = A[1].astype(jnp.int32)
    pad = E_PAD - E
    src = jnp.concatenate([src, jnp.zeros((pad,), jnp.int32)])
    dst = jnp.concatenate([dst, jnp.full((pad,), N, jnp.int32)])
    return src.reshape(NS, C_CHUNKS, CHUNK), dst.reshape(NS, C_CHUNKS, CHUNK)


def _make_sc_diffusion(R):
    mesh = plsc.VectorSubcoreMesh(core_axis_name="c", subcore_axis_name="s")
    ytype = jax.ShapeDtypeStruct((NA, R, N_PAD, F), jnp.float32)

    @functools.partial(
        pl.kernel,
        out_type=(ytype, ytype),
        mesh=mesh,
        scratch_types=[
            pltpu.VMEM((C_CHUNKS, CHUNK), jnp.int32),
            pltpu.VMEM((C_CHUNKS, CHUNK), jnp.int32),
            pltpu.VMEM((CHUNK, F), jnp.float32),
            pltpu.VMEM((ZR, F), jnp.float32),
            pltpu.VMEM_SHARED((N_PAD, F), jnp.float32),
            pltpu.SemaphoreType.DMA,
        ],
    )
    def sc_diffusion(x_hbm, se0, de0, se1, de1, zeros_hbm,
                     y1_hbm, y2_hbm,
                     sbuf, dbuf, rows, zbuf, acc, sem):
        cid = lax.axis_index("c")
        sid = lax.axis_index("s")
        base = sid * FR
        pltpu.sync_copy(zeros_hbm, zbuf)

        def one_pass(src_slab, out_slab):
            pltpu.sync_copy(zbuf, acc.at[pl.ds(base, ZR)])
            pltpu.sync_copy(zbuf, acc.at[pl.ds(base + ZR, ZR)])
            plsc.subcore_barrier()

            @pl.loop(0, C_CHUNKS)
            def _(j):
                pltpu.async_copy(src_slab.at[sbuf.at[j]], rows, sem).wait()
                pltpu.sync_copy(rows, acc.at[dbuf.at[j]], add=True)

            plsc.subcore_barrier()
            pltpu.sync_copy(acc.at[pl.ds(base, FR)], out_slab.at[pl.ds(base, FR)])
            plsc.subcore_barrier()

        for a in range(NA):
            s_e, d_e = (se0, de0) if a == 0 else (se1, de1)
            pltpu.sync_copy(s_e.at[sid], sbuf)
            pltpu.sync_copy(d_e.at[sid], dbuf)
            for rr in range(R // NC):
                r = rr * NC + cid
                one_pass(x_hbm.at[r], y1_hbm.at[a, r])
                one_pass(y1_hbm.at[a, r], y2_hbm.at[a, r])

    return sc_diffusion


_sc_diffusion_pre = _make_sc_diffusion(B * L)
_sc_diffusion_step = _make_sc_diffusion(B)


M_BLK = 2000


def _feat_specs():
    return [
        pl.BlockSpec((1, M_BLK, F), lambda g, i: (g, i, 0)),
        pl.BlockSpec((1, 1, M_BLK, F), lambda g, i: (0, g, i, 0)),
        pl.BlockSpec((1, 1, M_BLK, F), lambda g, i: (0, g, i, 0)),
        pl.BlockSpec((1, 1, M_BLK, F), lambda g, i: (1, g, i, 0)),
        pl.BlockSpec((1, 1, M_BLK, F), lambda g, i: (1, g, i, 0)),
    ]


def _matmul5(feats, W):
    acc = jnp.zeros((feats[0].shape[0], W.shape[-1]), jnp.float32)
    for m in range(5):
        acc = acc + jnp.dot(feats[m], W[m], preferred_element_type=jnp.float32)
    return acc


def _precompute_body(xr, y1a0, y2a0, y1a1, y2a1, wg, wc, bg, bc, xg_out, xc_out):
    feats = [xr[0], y1a0[0, 0], y2a0[0, 0], y1a1[0, 0], y2a1[0, 0]]
    xg_out[0, 0] = _matmul5(feats, wg[...]) + bg[0]
    xc_out[0, 0] = _matmul5(feats, wc[...]) + bc[0]


def _precompute_tc(xr, Y1, Y2, WgX, WcX, bg, bc):
    grid = (B * L, N // M_BLK)
    return pl.pallas_call(
        _precompute_body,
        grid=grid,
        in_specs=_feat_specs() + [
            pl.BlockSpec((5, F, 2 * F), lambda g, i: (0, 0, 0)),
            pl.BlockSpec((5, F, F), lambda g, i: (0, 0, 0)),
            pl.BlockSpec((1, 2 * F), lambda g, i: (0, 0)),
            pl.BlockSpec((1, F), lambda g, i: (0, 0)),
        ],
        out_specs=[
            pl.BlockSpec((1, 1, M_BLK, 2 * F), lambda g, i: (g // L, g % L, i, 0)),
            pl.BlockSpec((1, 1, M_BLK, F), lambda g, i: (g // L, g % L, i, 0)),
        ],
        out_shape=[
            jax.ShapeDtypeStruct((B, L, N, 2 * F), jnp.float32),
            jax.ShapeDtypeStruct((B, L, N, F), jnp.float32),
        ],
    )(xr, Y1, Y2, Y1, Y2, WgX, WcX, bg.reshape(1, -1), bc.reshape(1, -1))


def _gate_body(h, y1a0, y2a0, y1a1, y2a1, wg, xg, rh_out, u_out):
    hv = h[0]
    feats = [hv, y1a0[0, 0], y2a0[0, 0], y1a1[0, 0], y2a1[0, 0]]
    gate = jax.nn.sigmoid(_matmul5(feats, wg[...]) + xg[0, 0])
    rh_out[0] = gate[:, :F] * hv
    u_out[0] = gate[:, F:]


def _gate_tc(t, h, Y1, Y2, WgH, XG):
    grid = (B, N // M_BLK)
    return pl.pallas_call(
        _gate_body,
        grid=grid,
        in_specs=_feat_specs() + [
            pl.BlockSpec((5, F, 2 * F), lambda g, i: (0, 0, 0)),
            pl.BlockSpec((1, 1, M_BLK, 2 * F), lambda g, i, t=t: (g, t, i, 0)),
        ],
        out_specs=[
            pl.BlockSpec((1, M_BLK, F), lambda g, i: (g, i, 0)),
            pl.BlockSpec((1, M_BLK, F), lambda g, i: (g, i, 0)),
        ],
        out_shape=[
            jax.ShapeDtypeStruct((B, N, F), jnp.float32),
            jax.ShapeDtypeStruct((B, N, F), jnp.float32),
        ],
    )(h, Y1, Y2, Y1, Y2, WgH, XG)


def _cand_body(rh, y1a0, y2a0, y1a1, y2a1, wc, xc, u, h, h_out):
    feats = [rh[0], y1a0[0, 0], y2a0[0, 0], y1a1[0, 0], y2a1[0, 0]]
    c = jnp.tanh(_matmul5(feats, wc[...]) + xc[0, 0])
    uv = u[0]
    h_out[0] = uv * h[0] + (1.0 - uv) * c


def _cand_tc(t, rh, Y1, Y2, WcH, XC, u, h):
    grid = (B, N // M_BLK)
    return pl.pallas_call(
        _cand_body,
        grid=grid,
        in_specs=_feat_specs() + [
            pl.BlockSpec((5, F, F), lambda g, i: (0, 0, 0)),
            pl.BlockSpec((1, 1, M_BLK, F), lambda g, i, t=t: (g, t, i, 0)),
            pl.BlockSpec((1, M_BLK, F), lambda g, i: (g, i, 0)),
            pl.BlockSpec((1, M_BLK, F), lambda g, i: (g, i, 0)),
        ],
        out_specs=pl.BlockSpec((1, M_BLK, F), lambda g, i: (g, i, 0)),
        out_shape=jax.ShapeDtypeStruct((B, N, F), jnp.float32),
    )(rh, Y1, Y2, Y1, Y2, WcH, XC, u, h)


def kernel(x, A0, A1, W_gate, b_gate, W_cand, b_cand):
    WgH, WgX = _fold_weights(W_gate)
    WcH, WcX = _fold_weights(W_cand)
    se0, de0 = _prep_edges(A0)
    se1, de1 = _prep_edges(A1)
    zeros_tpl = jnp.zeros((ZR, F), jnp.float32)

    xr = x.reshape(B * L, N, F)
    XY1, XY2 = _sc_diffusion_pre(xr, se0, de0, se1, de1, zeros_tpl)
    XG, XC = _precompute_tc(xr, XY1, XY2, WgX, WcX, b_gate, b_cand)

    h = jnp.zeros((B, N, F), jnp.float32)
    zfeat = jnp.zeros((NA, B, N_PAD, F), jnp.float32)
    outs = []
    for t in range(L):
        if t == 0:
            hy1, hy2 = zfeat, zfeat
        else:
            hy1, hy2 = _sc_diffusion_step(h, se0, de0, se1, de1, zeros_tpl)
        rh, u = _gate_tc(t, h, hy1, hy2, WgH, XG)
        if t == 0:
            ry1, ry2 = zfeat, zfeat
        else:
            ry1, ry2 = _sc_diffusion_step(rh, se0, de0, se1, de1, zeros_tpl)
        h = _cand_tc(t, rh, ry1, ry2, WcH, XC, u, h)
        outs.append(h)

    return jnp.stack(outs, axis=1), h


# speedup vs baseline: 48.2667x; 48.2667x over previous
import functools

import jax
import jax.numpy as jnp
from jax import lax
from jax.experimental import pallas as pl
from jax.experimental.pallas import tpu as pltpu
from jax.experimental.pallas import tpu_sc as plsc

F = 128
B = 2
L = 8
N = 10000
E = 160000
NA = 2
NC = 2
NS = 16
CHUNK = 128
C_CHUNKS = -(-E // (NS * CHUNK))
E_PAD = NS * CHUNK * C_CHUNKS
N_PAD = 10112
FR = N_PAD // NS


def _split_weights(W):
    O = W.shape[1]
    Wm = W.reshape(5, 2 * F, O)
    return Wm[:, :F, :], Wm[:, F:, :]


def _prep_edges(A):
    src = A[0].astype(jnp.int32)
    dst = A[1].astype(jnp.int32)
    pad = E_PAD - E
    src = jnp.concatenate([src, jnp.zeros((pad,), jnp.int32)])
    dst = jnp.concatenate([dst, jnp.full((pad,), N, jnp.int32)])
    return src.reshape(NS, C_CHUNKS, CHUNK), dst.reshape(NS, C_CHUNKS, CHUNK)


def _make_sc_diffusion(R):
    mesh = plsc.VectorSubcoreMesh(
        core_axis_name="c", subcore_axis_name="s", num_cores=NC, num_subcores=NS
    )
    ytype = jax.ShapeDtypeStruct((NA, R, N_PAD, F), jnp.float32)

    @functools.partial(
        pl.kernel,
        out_type=(ytype, ytype),
        mesh=mesh,
        scratch_types=[
            pltpu.VMEM((C_CHUNKS, CHUNK), jnp.int32),
            pltpu.VMEM((C_CHUNKS, CHUNK), jnp.int32),
            pltpu.VMEM((CHUNK, F), jnp.float32),
            pltpu.VMEM_SHARED((N_PAD, F), jnp.float32),
            pltpu.SemaphoreType.DMA,
        ],
    )
    def sc_diffusion(x_hbm, se0, de0, se1, de1, zeros_hbm,
                     y1_hbm, y2_hbm,
                     sbuf, dbuf, rows, acc, sem):
        cid = lax.axis_index("c")
        sid = lax.axis_index("s")
        base = sid * FR

        def one_pass(src_slab, out_slab):
            pltpu.sync_copy(zeros_hbm, acc.at[pl.ds(base, FR)])
            plsc.subcore_barrier()

            @pl.loop(0, C_CHUNKS)
            def _(j):
                pltpu.async_copy(src_slab.at[sbuf.at[j]], rows, sem).wait()
                pltpu.sync_copy(rows, acc.at[dbuf.at[j]], add=True)

            plsc.subcore_barrier()
            pltpu.sync_copy(acc.at[pl.ds(base, FR)], out_slab.at[pl.ds(base, FR)])
            plsc.subcore_barrier()

        for a in range(NA):
            s_e, d_e = (se0, de0) if a == 0 else (se1, de1)
            pltpu.sync_copy(s_e.at[sid], sbuf)
            pltpu.sync_copy(d_e.at[sid], dbuf)
            for rr in range(R // NC):
                r = rr * NC + cid
                one_pass(x_hbm.at[r], y1_hbm.at[a, r])
                one_pass(y1_hbm.at[a, r], y2_hbm.at[a, r])

    return sc_diffusion


_sc_diffusion_pre = _make_sc_diffusion(B * L)
_sc_diffusion_step = _make_sc_diffusion(B)


M_BLK = 2000


def _feat_specs():
    return [
        pl.BlockSpec((1, M_BLK, F), lambda g, i: (g, i, 0)),
        pl.BlockSpec((1, 1, M_BLK, F), lambda g, i: (0, g, i, 0)),
        pl.BlockSpec((1, 1, M_BLK, F), lambda g, i: (0, g, i, 0)),
        pl.BlockSpec((1, 1, M_BLK, F), lambda g, i: (1, g, i, 0)),
        pl.BlockSpec((1, 1, M_BLK, F), lambda g, i: (1, g, i, 0)),
    ]


def _cheb_feats(base, y1a0, y2a0, y1a1, y2a1):
    return [base, y1a0, 2.0 * y2a0 - base, y1a1, 2.0 * y2a1 - base]


def _matmul5(feats, W):
    acc = jnp.zeros((feats[0].shape[0], W.shape[-1]), jnp.float32)
    for m in range(5):
        acc = acc + jnp.dot(feats[m], W[m], precision=lax.Precision.DEFAULT,
                            preferred_element_type=jnp.float32)
    return acc


def _precompute_body(xr, y1a0, y2a0, y1a1, y2a1, wg, wc, bg, bc, xg_out, xc_out):
    feats = _cheb_feats(xr[0], y1a0[0, 0], y2a0[0, 0], y1a1[0, 0], y2a1[0, 0])
    xg_out[0, 0] = _matmul5(feats, wg[...]) + bg[0]
    xc_out[0, 0] = _matmul5(feats, wc[...]) + bc[0]


def _precompute_tc(xr, Y1, Y2, WgX, WcX, bg, bc):
    grid = (B * L, N // M_BLK)
    return pl.pallas_call(
        _precompute_body,
        grid=grid,
        in_specs=_feat_specs() + [
            pl.BlockSpec((5, F, 2 * F), lambda g, i: (0, 0, 0)),
            pl.BlockSpec((5, F, F), lambda g, i: (0, 0, 0)),
            pl.BlockSpec((1, 2 * F), lambda g, i: (0, 0)),
            pl.BlockSpec((1, F), lambda g, i: (0, 0)),
        ],
        out_specs=[
            pl.BlockSpec((1, 1, M_BLK, 2 * F), lambda g, i: (g // L, g % L, i, 0)),
            pl.BlockSpec((1, 1, M_BLK, F), lambda g, i: (g // L, g % L, i, 0)),
        ],
        out_shape=[
            jax.ShapeDtypeStruct((B, L, N, 2 * F), jnp.float32),
            jax.ShapeDtypeStruct((B, L, N, F), jnp.float32),
        ],
    )(xr, Y1, Y2, Y1, Y2, WgX, WcX, bg.reshape(1, -1), bc.reshape(1, -1))


def _gate_body(h, y1a0, y2a0, y1a1, y2a1, wg, xg, pre_out):
    feats = _cheb_feats(h[0], y1a0[0, 0], y2a0[0, 0], y1a1[0, 0], y2a1[0, 0])
    pre_out[0] = _matmul5(feats, wg[...]) + xg[0, 0]


def _gate_tc(t, h, Y1, Y2, WgH, XG):
    grid = (B, N // M_BLK)
    return pl.pallas_call(
        _gate_body,
        grid=grid,
        in_specs=_feat_specs() + [
            pl.BlockSpec((5, F, 2 * F), lambda g, i: (0, 0, 0)),
            pl.BlockSpec((1, 1, M_BLK, 2 * F), lambda g, i, t=t: (g, t, i, 0)),
        ],
        out_specs=pl.BlockSpec((1, M_BLK, 2 * F), lambda g, i: (g, i, 0)),
        out_shape=jax.ShapeDtypeStruct((B, N, 2 * F), jnp.float32),
    )(h, Y1, Y2, Y1, Y2, WgH, XG)


def _cand_body(rh, y1a0, y2a0, y1a1, y2a1, wc, xc, pre_out):
    feats = _cheb_feats(rh[0], y1a0[0, 0], y2a0[0, 0], y1a1[0, 0], y2a1[0, 0])
    pre_out[0] = _matmul5(feats, wc[...]) + xc[0, 0]


def _cand_tc(t, rh, Y1, Y2, WcH, XC):
    grid = (B, N // M_BLK)
    return pl.pallas_call(
        _cand_body,
        grid=grid,
        in_specs=_feat_specs() + [
            pl.BlockSpec((5, F, F), lambda g, i: (0, 0, 0)),
            pl.BlockSpec((1, 1, M_BLK, F), lambda g, i, t=t: (g, t, i, 0)),
        ],
        out_specs=pl.BlockSpec((1, M_BLK, F), lambda g, i: (g, i, 0)),
        out_shape=jax.ShapeDtypeStruct((B, N, F), jnp.float32),
    )(rh, Y1, Y2, Y1, Y2, WcH, XC)


def kernel(x, A0, A1, W_gate, b_gate, W_cand, b_cand):
    WgH, WgX = _split_weights(W_gate)
    WcH, WcX = _split_weights(W_cand)
    se0, de0 = _prep_edges(A0)
    se1, de1 = _prep_edges(A1)
    zeros_tpl = jnp.zeros((FR, F), jnp.float32)

    xr = x.reshape(B * L, N, F)
    XY1, XY2 = _sc_diffusion_pre(xr, se0, de0, se1, de1, zeros_tpl)
    XG, XC = _precompute_tc(xr, XY1, XY2, WgX, WcX, b_gate, b_cand)

    h = jnp.zeros((B, N, F), jnp.float32)
    zfeat = jnp.zeros((NA, B, N_PAD, F), jnp.float32)
    outs = []
    for t in range(L):
        if t == 0:
            hy1, hy2 = zfeat, zfeat
        else:
            hy1, hy2 = _sc_diffusion_step(h, se0, de0, se1, de1, zeros_tpl)
        gate = jax.nn.sigmoid(_gate_tc(t, h, hy1, hy2, WgH, XG))
        r_t, u_t = gate[..., :F], gate[..., F:]
        rh = r_t * h
        if t == 0:
            ry1, ry2 = zfeat, zfeat
        else:
            ry1, ry2 = _sc_diffusion_step(rh, se0, de0, se1, de1, zeros_tpl)
        c_t = jnp.tanh(_cand_tc(t, rh, ry1, ry2, WcH, XC))
        h = u_t * h + (1.0 - u_t) * c_t
        outs.append(h)

    return jnp.stack(outs, axis=1), h
